# baseline (speedup 1.0000x reference)
import functools

import jax
import jax.numpy as jnp
from jax import lax
from jax.experimental import pallas as pl
from jax.experimental.pallas import tpu as pltpu
from jax.experimental.pallas import tpu_sc as plsc

_SEQ = 200
_BATCH = 4096
_DIM = 64
_NC = 2
_NS = 16
_NW = _NC * _NS
_TOTAL_ROWS = _BATCH * _SEQ
_ROWS_PER_W = _TOTAL_ROWS // _NW
_SEQS_PER_W = _ROWS_PER_W // _SEQ
_LANES = 16
_VREGS_PER_ROW = _DIM // _LANES


def _make_sc_call():
    mesh = plsc.VectorSubcoreMesh(
        core_axis_name="c", subcore_axis_name="s",
        num_cores=_NC, num_subcores=_NS)

    @functools.partial(
        pl.kernel,
        out_type=jax.ShapeDtypeStruct((_TOTAL_ROWS, _DIM), jnp.float32),
        mesh=mesh,
        scratch_types=[
            pltpu.VMEM((2, _SEQ), jnp.int32),
            pltpu.VMEM((2, _SEQ, _DIM), jnp.float32),
            pltpu.SemaphoreType.DMA((2,)),
            pltpu.SemaphoreType.DMA((2,)),
        ],
        compiler_params=pltpu.CompilerParams(use_tc_tiling_on_sc=False),
    )
    def sc_embed(x_hbm, tok_hbm, pos_hbm, out_hbm, idx_v, rows_v,
                 gsem, osem):
        wid = lax.axis_index("s") * _NC + lax.axis_index("c")
        base0 = wid * _ROWS_PER_W

        pltpu.sync_copy(x_hbm.at[pl.ds(base0, _SEQ)], idx_v.at[0])
        pltpu.sync_copy(pos_hbm, rows_v.at[0])
        pltpu.async_copy(tok_hbm.at[idx_v.at[0]], rows_v.at[0], gsem.at[0],
                         add=True)

        def chunk_body(i, carry):
            p = lax.rem(i, 2)
            q = 1 - p

            @pl.when(i + 1 < _SEQS_PER_W)
            def _():
                @pl.when(i >= 1)
                def _():
                    pltpu.make_async_copy(
                        rows_v.at[q],
                        out_hbm.at[pl.ds(base0, _SEQ)],
                        osem.at[q]).wait()
                nbase = base0 + (i + 1) * _SEQ
                pltpu.sync_copy(x_hbm.at[pl.ds(nbase, _SEQ)], idx_v.at[q])
                pltpu.sync_copy(pos_hbm, rows_v.at[q])
                pltpu.async_copy(tok_hbm.at[idx_v.at[q]], rows_v.at[q],
                                 gsem.at[q], add=True)

            pltpu.make_async_copy(tok_hbm.at[idx_v.at[p]], rows_v.at[p],
                                  gsem.at[p]).wait()

            pltpu.async_copy(rows_v.at[p],
                             out_hbm.at[pl.ds(base0 + i * _SEQ, _SEQ)],
                             osem.at[p])
            return carry

        lax.fori_loop(0, _SEQS_PER_W, chunk_body, 0)

        for p in range(2):
            pltpu.make_async_copy(rows_v.at[p],
                                  out_hbm.at[pl.ds(base0, _SEQ)],
                                  osem.at[p]).wait()

    return sc_embed


_sc_embed = _make_sc_call()


@jax.jit
def kernel(x, token_table, pos_table):
    x_flat = x.reshape(-1)
    out = _sc_embed(x_flat, token_table, pos_table)
    return out.reshape(_BATCH, _SEQ, _DIM)


# speedup vs baseline: 1.1709x; 1.1709x over previous
import functools

import jax
import jax.numpy as jnp
from jax import lax
from jax.experimental import pallas as pl
from jax.experimental.pallas import tpu as pltpu
from jax.experimental.pallas import tpu_sc as plsc

_SEQ = 200
_BATCH = 4096
_DIM = 64
_NC = 2
_NS = 16
_NW = _NC * _NS
_TOTAL_ROWS = _BATCH * _SEQ
_ROWS_PER_W = _TOTAL_ROWS // _NW
_SEQS_PER_W = _ROWS_PER_W // _SEQ
_LANES = 16
_VREGS_PER_ROW = _DIM // _LANES


def _make_sc_call():
    mesh = plsc.VectorSubcoreMesh(
        core_axis_name="c", subcore_axis_name="s",
        num_cores=_NC, num_subcores=_NS)

    @functools.partial(
        pl.kernel,
        out_type=jax.ShapeDtypeStruct((_TOTAL_ROWS, _DIM), jnp.float32),
        mesh=mesh,
        scratch_types=[
            pltpu.VMEM((2, _SEQ), jnp.int32),
            pltpu.VMEM((2, _SEQ, _DIM), jnp.float32),
            pltpu.VMEM((_SEQ, _DIM), jnp.float32),
            pltpu.SemaphoreType.DMA((2,)),
            pltpu.SemaphoreType.DMA((2,)),
        ],
        compiler_params=pltpu.CompilerParams(use_tc_tiling_on_sc=False),
    )
    def sc_embed(x_hbm, tok_hbm, pos_hbm, out_hbm, idx_v, rows_v, pos_v,
                 gsem, osem):
        wid = lax.axis_index("s") * _NC + lax.axis_index("c")
        base0 = wid * _ROWS_PER_W
        pltpu.sync_copy(pos_hbm, pos_v)

        pltpu.sync_copy(x_hbm.at[pl.ds(base0, _SEQ)], idx_v.at[0])
        pltpu.async_copy(tok_hbm.at[idx_v.at[0]], rows_v.at[0], gsem.at[0])

        def chunk_body(i, carry):
            p = lax.rem(i, 2)
            q = 1 - p

            @pl.when(i + 1 < _SEQS_PER_W)
            def _():
                @pl.when(i >= 1)
                def _():
                    pltpu.make_async_copy(
                        rows_v.at[q],
                        out_hbm.at[pl.ds(base0, _SEQ)],
                        osem.at[q]).wait()
                nbase = base0 + (i + 1) * _SEQ
                pltpu.sync_copy(x_hbm.at[pl.ds(nbase, _SEQ)], idx_v.at[q])
                pltpu.async_copy(tok_hbm.at[idx_v.at[q]], rows_v.at[q],
                                 gsem.at[q])

            pltpu.make_async_copy(tok_hbm.at[idx_v.at[p]], rows_v.at[p],
                                  gsem.at[p]).wait()

            @plsc.parallel_loop(0, _SEQ, 1, unroll=4)
            def _(r):
                for c in range(_VREGS_PER_ROW):
                    s = pl.ds(c * _LANES, _LANES)
                    plsc.addupdate(rows_v.at[p, r, s], pos_v[r, s])

            pltpu.async_copy(rows_v.at[p],
                             out_hbm.at[pl.ds(base0 + i * _SEQ, _SEQ)],
                             osem.at[p])
            return carry

        lax.fori_loop(0, _SEQS_PER_W, chunk_body, 0)

        for p in range(2):
            pltpu.make_async_copy(rows_v.at[p],
                                  out_hbm.at[pl.ds(base0, _SEQ)],
                                  osem.at[p]).wait()

    return sc_embed


_sc_embed = _make_sc_call()


@jax.jit
def kernel(x, token_table, pos_table):
    x_flat = x.reshape(-1)
    out = _sc_embed(x_flat, token_table, pos_table)
    return out.reshape(_BATCH, _SEQ, _DIM)
